# baseline (speedup 1.0000x reference)
import jax
import jax.numpy as jnp
from jax import lax
from jax.experimental import pallas as pl
from jax.experimental.pallas import tpu as pltpu
from jax.experimental.pallas import tpu_sc as plsc

_N = 100000
_G = 16
_NS = 16
_STRIDE = 6240
_CHUNK = 6400
_VECS = _CHUNK // 16


def _fused_body(batch_hbm, ctab_hbm, out_hbm,
                chunk_v, pres_v, ctab_v, presall_v, out_v, zero_v, shared):
    s = lax.axis_index("s")

    @pl.when(s == 0)
    def _():
        zero_v[...] = jnp.zeros((16,), jnp.float32)
        pltpu.sync_copy(zero_v, shared)
        pltpu.sync_copy(ctab_hbm, ctab_v)

    pltpu.sync_copy(batch_hbm.at[pl.ds(s * _STRIDE, _CHUNK)], chunk_v)
    plsc.subcore_barrier()

    pres_v[...] = jnp.zeros((16,), jnp.float32)
    ones = jnp.ones((16,), jnp.float32)

    def step(i, carry):
        idx = chunk_v[pl.ds(i * 16, 16)]
        plsc.store_scatter(pres_v, [idx], ones)
        return carry

    lax.fori_loop(0, _VECS, step, 0)
    pltpu.sync_copy(pres_v, shared.at[jnp.arange(16, dtype=jnp.int32)], add=True)
    plsc.subcore_barrier()

    @pl.when(s == 0)
    def _():
        pltpu.sync_copy(shared, presall_v)
        pv = presall_v[...]
        c0 = ctab_v[pl.ds(0, 16)]
        c1 = ctab_v[pl.ds(16, 16)]
        for seg in range(_G):
            p = pv[seg]

            @pl.when(p > 0.0)
            def _():
                out_v[pl.ds(16 * seg, 16)] = c0

            @pl.when(p <= 0.0)
            def _():
                out_v[pl.ds(16 * seg, 16)] = c1

        pltpu.sync_copy(out_v, out_hbm)


_fused_sc = pl.kernel(
    _fused_body,
    out_type=jax.ShapeDtypeStruct((_G * 16,), jnp.float32),
    mesh=plsc.VectorSubcoreMesh(core_axis_name="c", subcore_axis_name="s"),
    scratch_types=[
        pltpu.VMEM((_CHUNK,), jnp.int32),
        pltpu.VMEM((16,), jnp.float32),
        pltpu.VMEM((32,), jnp.float32),
        pltpu.VMEM((16,), jnp.float32),
        pltpu.VMEM((_G * 16,), jnp.float32),
        pltpu.VMEM((16,), jnp.float32),
        pltpu.VMEM_SHARED((16,), jnp.float32),
    ],
    compiler_params=pltpu.CompilerParams(needs_layout_passes=False),
)


def _table_kernel(wfc1_ref, bfc1_ref, wfc2_ref, bfc2_ref,
                  wlab_ref, blab_ref, wbb_ref, bbb_ref, ctab_ref):
    zero = jnp.zeros((1, 256), jnp.float32)
    ninf = jnp.full((1, 256), -jnp.inf, jnp.float32)
    g2 = jnp.concatenate([zero, ninf], axis=0)
    h = jnp.maximum(jnp.dot(g2, wfc1_ref[...],
                            preferred_element_type=jnp.float32) + bfc1_ref[...], 0.0)
    h = jnp.maximum(jnp.dot(h, wfc2_ref[...],
                            preferred_element_type=jnp.float32) + bfc2_ref[...], 0.0)
    lab = jnp.dot(h, wlab_ref[...],
                  preferred_element_type=jnp.float32) + blab_ref[...]
    bb = jnp.dot(h, wbb_ref[...],
                 preferred_element_type=jnp.float32) + bbb_ref[...]
    ctab_ref[...] = jnp.concatenate([lab, bb], axis=1)


def kernel(pos, batch, W1c1, b1c1, W2c1, b2c1, W1c2, b1c2, W2c2, b2c2,
           Wfc1, bfc1, Wfc2, bfc2, Wlab, blab, Wbb, bbb):
    ctab = pl.pallas_call(
        _table_kernel,
        out_shape=jax.ShapeDtypeStruct((2, 16), jnp.float32),
    )(Wfc1, bfc1.reshape(1, 256), Wfc2, bfc2.reshape(1, 128),
      Wlab, blab.reshape(1, 10), Wbb, bbb.reshape(1, 6))
    out = _fused_sc(batch, ctab.reshape(32))
    out = out.reshape(_G, 16)
    return (out[:, :10], out[:, 10:16])
